# baseline (speedup 1.0000x reference)
import functools

import jax
import jax.numpy as jnp
from jax import lax
from jax.experimental import pallas as pl
from jax.experimental.pallas import tpu as pltpu
from jax.experimental.pallas import tpu_sc as plsc

EMBED = 64
SEQ = 512
EPS = 1e-5
NW = 32
CHUNK = 128
LANES = 16
VPR = EMBED // LANES


def _fast_rsqrt(x):
    i = lax.bitcast_convert_type(x, jnp.int32)
    i = jnp.int32(0x5F3759DF) - (i >> 1)
    y = lax.bitcast_convert_type(i, jnp.float32)
    half = jnp.float32(0.5) * x
    for _ in range(3):
        y = y * (jnp.float32(1.5) - half * y * y)
    return y


_GATHER_DNUMS = lax.GatherDimensionNumbers(
    offset_dims=(), collapsed_slice_dims=(0,), start_index_map=(0,))


def _shuffle(v, idx):
    return lax.gather(v, idx[:, None], _GATHER_DNUMS, slice_sizes=(1,),
                      mode=lax.GatherScatterMode.PROMISE_IN_BOUNDS)


def _lane_total(v, shuffles):
    for idx in shuffles:
        v = v + _shuffle(v, idx)
    return v


def _make_emb_kernel(n_rows):
    rows_per_w = n_rows // NW
    chunks_per_w = rows_per_w // CHUNK
    pos_chunks = SEQ // CHUNK

    mesh = plsc.VectorSubcoreMesh(core_axis_name="c", subcore_axis_name="s")

    @functools.partial(
        pl.kernel,
        mesh=mesh,
        compiler_params=pltpu.CompilerParams(use_tc_tiling_on_sc=False),
        out_type=jax.ShapeDtypeStruct((n_rows, EMBED), jnp.float32),
        scratch_types=[
            pltpu.VMEM((SEQ, EMBED), jnp.float32),
            pltpu.VMEM((EMBED,), jnp.float32),
            pltpu.VMEM((EMBED,), jnp.float32),
            pltpu.VMEM((CHUNK,), jnp.int32),
            pltpu.VMEM((CHUNK, EMBED), jnp.float32),
            pltpu.SemaphoreType.DMA,
        ],
    )
    def emb(ids_hbm, tok_hbm, pos_hbm, gamma_hbm, beta_hbm, out_hbm,
            pos_v, g_v, b_v, idx_v, rows_v, sem):
        wid = lax.axis_index("s") * 2 + lax.axis_index("c")
        base = wid * rows_per_w

        pltpu.sync_copy(pos_hbm, pos_v)
        pltpu.sync_copy(gamma_hbm, g_v)
        pltpu.sync_copy(beta_hbm, b_v)

        g = [g_v[pl.ds(LANES * j, LANES)] for j in range(VPR)]
        b = [b_v[pl.ds(LANES * j, LANES)] for j in range(VPR)]
        inv_d = jnp.float32(1.0 / EMBED)
        lane = lax.iota(jnp.int32, LANES)
        shuffles = [jnp.bitwise_xor(lane, jnp.int32(k)) for k in (8, 4, 2, 1)]

        def chunk_body(c, _):
            off = base + c * CHUNK
            pltpu.sync_copy(ids_hbm.at[pl.ds(off, CHUNK)], idx_v)
            pltpu.async_copy(tok_hbm.at[idx_v], rows_v, sem).wait()
            pos_off = (c % pos_chunks) * CHUNK

            def row_body(r, _):
                x = []
                for j in range(VPR):
                    t = rows_v[r, pl.ds(LANES * j, LANES)]
                    p = pos_v[pos_off + r, pl.ds(LANES * j, LANES)]
                    x.append(t + p)
                s = (x[0] + x[1]) + (x[2] + x[3])
                q = (x[0] * x[0] + x[1] * x[1]) + (x[2] * x[2] + x[3] * x[3])
                mean = _lane_total(s, shuffles) * inv_d
                var = _lane_total(q, shuffles) * inv_d - mean * mean
                inv = _fast_rsqrt(var + jnp.float32(EPS))
                for j in range(VPR):
                    rows_v[r, pl.ds(LANES * j, LANES)] = (
                        (x[j] - mean) * inv * g[j] + b[j])
                return 0

            lax.fori_loop(0, CHUNK, row_body, 0, unroll=2)
            pltpu.sync_copy(rows_v, out_hbm.at[pl.ds(off, CHUNK)])
            return 0

        lax.fori_loop(0, chunks_per_w, chunk_body, 0)

    return emb


def kernel(input_ids, token_table, pos_table, gamma, beta):
    batch, seq = input_ids.shape
    n_rows = batch * seq
    ids = input_ids.reshape(n_rows)
    emb = _make_emb_kernel(n_rows)
    out = emb(ids, token_table, pos_table, gamma, beta)
    return out.reshape(batch, seq, EMBED)


# speedup vs baseline: 1.3720x; 1.3720x over previous
import functools

import jax
import jax.numpy as jnp
from jax import lax
from jax.experimental import pallas as pl
from jax.experimental.pallas import tpu as pltpu
from jax.experimental.pallas import tpu_sc as plsc

EMBED = 64
SEQ = 512
EPS = 1e-5
NW = 32
CHUNK = 512
BB = 8


def _make_gather(n_rows):
    rows_per_w = n_rows // NW
    n_pairs = rows_per_w // (2 * CHUNK)

    mesh = plsc.VectorSubcoreMesh(core_axis_name="c", subcore_axis_name="s")

    @functools.partial(
        pl.kernel,
        mesh=mesh,
        compiler_params=pltpu.CompilerParams(use_tc_tiling_on_sc=False),
        out_type=jax.ShapeDtypeStruct((n_rows, EMBED), jnp.float32),
        scratch_types=[
            pltpu.VMEM((CHUNK,), jnp.int32),
            pltpu.VMEM((CHUNK,), jnp.int32),
            pltpu.VMEM((CHUNK, EMBED), jnp.float32),
            pltpu.VMEM((CHUNK, EMBED), jnp.float32),
            pltpu.SemaphoreType.DMA,
            pltpu.SemaphoreType.DMA,
        ],
    )
    def gather(ids_hbm, tok_hbm, out_hbm, idx0, idx1, rows0, rows1,
               sem0, sem1):
        wid = lax.axis_index("s") * 2 + lax.axis_index("c")
        base = wid * rows_per_w

        def body(i, _):
            off0 = base + i * (2 * CHUNK)
            off1 = off0 + CHUNK
            pltpu.sync_copy(ids_hbm.at[pl.ds(off0, CHUNK)], idx0)
            h0 = pltpu.async_copy(tok_hbm.at[idx0], rows0, sem0)
            pltpu.sync_copy(ids_hbm.at[pl.ds(off1, CHUNK)], idx1)
            h1 = pltpu.async_copy(tok_hbm.at[idx1], rows1, sem1)
            h0.wait()
            pltpu.sync_copy(rows0, out_hbm.at[pl.ds(off0, CHUNK)])
            h1.wait()
            pltpu.sync_copy(rows1, out_hbm.at[pl.ds(off1, CHUNK)])
            return 0

        lax.fori_loop(0, n_pairs, body, 0)

    return gather


def _ln_body(x_ref, pos_ref, g_ref, b_ref, o_ref):
    x = x_ref[...] + pos_ref[...][None, :, :]
    mean = jnp.mean(x, axis=-1, keepdims=True)
    var = jnp.mean(x * x, axis=-1, keepdims=True) - mean * mean
    inv = lax.rsqrt(var + EPS)
    o_ref[...] = (x - mean) * inv * g_ref[...] + b_ref[...]


def kernel(input_ids, token_table, pos_table, gamma, beta):
    batch, seq = input_ids.shape
    n_rows = batch * seq
    ids = input_ids.reshape(n_rows)

    gathered = _make_gather(n_rows)(ids, token_table)
    gathered = gathered.reshape(batch, seq, EMBED)

    out = pl.pallas_call(
        _ln_body,
        grid=(batch // BB,),
        in_specs=[
            pl.BlockSpec((BB, seq, EMBED), lambda i: (i, 0, 0)),
            pl.BlockSpec((seq, EMBED), lambda i: (0, 0)),
            pl.BlockSpec((1, EMBED), lambda i: (0, 0)),
            pl.BlockSpec((1, EMBED), lambda i: (0, 0)),
        ],
        out_specs=pl.BlockSpec((BB, seq, EMBED), lambda i: (i, 0, 0)),
        out_shape=jax.ShapeDtypeStruct((batch, seq, EMBED), jnp.float32),
    )(gathered, pos_table, gamma.reshape(1, EMBED), beta.reshape(1, EMBED))
    return out
